# baseline (speedup 1.0000x reference)
import functools

import jax
import jax.numpy as jnp
from jax import lax
from jax.experimental import pallas as pl
from jax.experimental.pallas import tpu as pltpu
from jax.experimental.pallas import tpu_sc as plsc

UNI = 100000
BUCKETS = 800000
EMBED = 300
BATCH = 1024
SEQ = 500
PAD_D = 16
SEQ_PAD = 512
CHUNK = 512
NC, NS = 2, 16
NW = NC * NS
EX_PER_W = BATCH // NW
CPE = SEQ_PAD // CHUNK
CPW = EX_PER_W * CPE
NBUF = 4

MM_BLK = 8192
NU_BLKS = (UNI + MM_BLK - 1) // MM_BLK
BI_BASE = NU_BLKS * MM_BLK
IDX_OFF = BI_BASE - UNI
N_TOT = BI_BASE + BUCKETS
MM_GRID = (N_TOT + MM_BLK - 1) // MM_BLK


def _mm_body(xu_ref, xb_ref, w_ref, o_ref):
    pid = pl.program_id(0)

    def emit(x_ref, zero_tail):
        yt = lax.dot_general(
            w_ref[...],
            x_ref[...],
            (((0,), (0,)), ((), ())),
            preferred_element_type=jnp.float32,
        )
        col = lax.broadcasted_iota(jnp.int32, yt.shape, 0)
        yt = jnp.where(col == 2, 1.0, yt)
        if zero_tail:
            row = pid * MM_BLK + lax.broadcasted_iota(jnp.int32, yt.shape, 1)
            yt = jnp.where((row == 0) | (row >= UNI), 0.0, yt)
        o_ref[:, :PAD_D] = yt.T

    @pl.when(pid < NU_BLKS)
    def _():
        emit(xu_ref, True)

    @pl.when(pid >= NU_BLKS)
    def _():
        emit(xb_ref, False)


def _tables_times_w(uni_t, bi_t, w_pad):
    return pl.pallas_call(
        _mm_body,
        grid=(MM_GRID,),
        in_specs=[
            pl.BlockSpec(
                (EMBED, MM_BLK), lambda i: (0, jnp.minimum(i, NU_BLKS - 1))
            ),
            pl.BlockSpec(
                (EMBED, MM_BLK), lambda i: (0, jnp.maximum(i - NU_BLKS, 0))
            ),
            pl.BlockSpec((EMBED, PAD_D), lambda i: (0, 0)),
        ],
        out_specs=pl.BlockSpec((MM_BLK, 128), lambda i: (i, 0)),
        out_shape=jax.ShapeDtypeStruct((N_TOT, 128), jnp.float32),
    )(uni_t, bi_t, w_pad)


def _pool(tablew, tok3, bias16):

    @functools.partial(
        pl.kernel,
        out_type=jax.ShapeDtypeStruct((BATCH * PAD_D,), jnp.float32),
        mesh=plsc.VectorSubcoreMesh(core_axis_name="c", subcore_axis_name="s"),
        compiler_params=pltpu.CompilerParams(use_tc_tiling_on_sc=False),
        scratch_types=(
            [
                pltpu.VMEM((CPW * CHUNK,), jnp.int32),
                pltpu.VMEM((EX_PER_W * PAD_D,), jnp.float32),
                pltpu.VMEM((PAD_D,), jnp.float32),
            ]
            + [pltpu.VMEM((CHUNK, PAD_D), jnp.float32) for _ in range(NBUF)]
            + [pltpu.SemaphoreType.DMA for _ in range(NBUF)]
        ),
    )
    def k(tw_hbm, tok_hbm, bias_hbm, out_hbm, idx_v, out_buf, bias_v, *rest):
        rows = rest[:NBUF]
        sems = rest[NBUF:]
        wid = lax.axis_index("s") * NC + lax.axis_index("c")
        pltpu.sync_copy(tok_hbm.at[wid], idx_v)
        pltpu.sync_copy(bias_hbm, bias_v)
        bias_vec = bias_v[...]
        lane = lax.broadcasted_iota(jnp.int32, (PAD_D,), 0)

        def remap(j, carry):
            v = idx_v[pl.ds(j * 16, 16)]
            idx_v[pl.ds(j * 16, 16)] = (
                jnp.where(v < UNI, v, v + IDX_OFF) * 8
            )
            return carry

        lax.fori_loop(0, (CPW * CHUNK) // 16, remap, 0)

        def fire(g):
            return pltpu.async_copy(
                tw_hbm.at[idx_v.at[pl.ds(g * CHUNK, CHUNK)]],
                rows[g % NBUF],
                sems[g % NBUF],
            )

        handles = [fire(g) for g in range(NBUF - 1)]
        acc = jnp.zeros((PAD_D,), jnp.float32)
        for g in range(CPW):
            if g + NBUF - 1 < CPW:
                handles.append(fire(g + NBUF - 1))
            handles[g].wait()
            rbuf = rows[g % NBUF]
            acc = lax.fori_loop(
                0, CHUNK, lambda r, a: a + rbuf[r], acc, unroll=8
            )
            if g % CPE == CPE - 1:
                e = g // CPE
                cnt = jnp.broadcast_to(acc[2], (PAD_D,))
                res = acc / cnt + bias_vec
                out_buf[pl.ds(e * PAD_D, PAD_D)] = res
                acc = jnp.zeros((PAD_D,), jnp.float32)
        pltpu.sync_copy(
            out_buf,
            out_hbm.at[pl.ds(wid * (EX_PER_W * PAD_D), EX_PER_W * PAD_D)],
        )

    return k(tablew, tok3, bias16)


def kernel(inputs, unigram, bigram, W, b):
    inputs = inputs.astype(jnp.int32)
    w_pad = jnp.zeros((EMBED, PAD_D), jnp.float32).at[:, :2].set(W)
    tablew = _tables_times_w(unigram.T, bigram.T, w_pad).reshape(
        N_TOT * 8, PAD_D
    )
    tok = jnp.pad(inputs, ((0, 0), (0, SEQ_PAD - SEQ)))
    tok2 = tok.reshape(NW, CPW * CHUNK)
    b16 = jnp.zeros((PAD_D,), jnp.float32).at[:2].set(b)
    out = _pool(tablew, tok2, b16)
    return out.reshape(BATCH, PAD_D)[:, :2]


# speedup vs baseline: 1.0089x; 1.0089x over previous
import functools

import jax
import jax.numpy as jnp
from jax import lax
from jax.experimental import pallas as pl
from jax.experimental.pallas import tpu as pltpu
from jax.experimental.pallas import tpu_sc as plsc

UNI = 100000
BUCKETS = 800000
EMBED = 300
BATCH = 1024
SEQ = 500
PAD_D = 16
SEQ_PAD = 512
CHUNK = 512
NC, NS = 2, 16
NW = NC * NS
EX_PER_W = BATCH // NW
CPE = SEQ_PAD // CHUNK
CPW = EX_PER_W * CPE
NBUF = 4

MM_BLK = 8192
NU_BLKS = (UNI + MM_BLK - 1) // MM_BLK
BI_BASE = NU_BLKS * MM_BLK
IDX_OFF = BI_BASE - UNI
N_TOT = BI_BASE + BUCKETS
MM_GRID = (N_TOT + MM_BLK - 1) // MM_BLK


def _mm_body(xu_ref, xb_ref, w_ref, o_ref):
    pid = pl.program_id(0)

    def emit(x_ref, zero_tail):
        yt = lax.dot_general(
            w_ref[...],
            x_ref[...],
            (((0,), (0,)), ((), ())),
            preferred_element_type=jnp.float32,
        )
        col = lax.broadcasted_iota(jnp.int32, yt.shape, 0)
        yt = jnp.where(col == 2, 1.0, yt)
        if zero_tail:
            row = pid * MM_BLK + lax.broadcasted_iota(jnp.int32, yt.shape, 1)
            yt = jnp.where((row == 0) | (row >= UNI), 0.0, yt)
        o_ref[:, :PAD_D] = yt.T

    @pl.when(pid < NU_BLKS)
    def _():
        emit(xu_ref, True)

    @pl.when(pid >= NU_BLKS)
    def _():
        emit(xb_ref, False)


def _tables_times_w(uni_t, bi_t, w_pad):
    return pl.pallas_call(
        _mm_body,
        grid=(MM_GRID,),
        in_specs=[
            pl.BlockSpec(
                (EMBED, MM_BLK), lambda i: (0, jnp.minimum(i, NU_BLKS - 1))
            ),
            pl.BlockSpec(
                (EMBED, MM_BLK), lambda i: (0, jnp.maximum(i - NU_BLKS, 0))
            ),
            pl.BlockSpec((EMBED, PAD_D), lambda i: (0, 0)),
        ],
        out_specs=pl.BlockSpec((MM_BLK, 128), lambda i: (i, 0)),
        out_shape=jax.ShapeDtypeStruct((N_TOT, 128), jnp.float32),
    )(uni_t, bi_t, w_pad)


def _pool(tablew, tok3, bias16):

    @functools.partial(
        pl.kernel,
        out_type=jax.ShapeDtypeStruct((BATCH * PAD_D,), jnp.float32),
        mesh=plsc.VectorSubcoreMesh(core_axis_name="c", subcore_axis_name="s"),
        compiler_params=pltpu.CompilerParams(use_tc_tiling_on_sc=False),
        scratch_types=(
            [
                pltpu.VMEM((CPW * CHUNK,), jnp.int32),
                pltpu.VMEM((EX_PER_W * PAD_D,), jnp.float32),
                pltpu.VMEM((PAD_D,), jnp.float32),
            ]
            + [pltpu.VMEM((CHUNK, PAD_D), jnp.float32) for _ in range(NBUF)]
            + [pltpu.SemaphoreType.DMA for _ in range(NBUF)]
        ),
    )
    def k(tw_hbm, tok_hbm, bias_hbm, out_hbm, idx_v, out_buf, bias_v, *rest):
        rows = rest[:NBUF]
        sems = rest[NBUF:]
        wid = lax.axis_index("s") * NC + lax.axis_index("c")
        pltpu.sync_copy(tok_hbm.at[wid], idx_v)
        pltpu.sync_copy(bias_hbm, bias_v)
        bias_vec = bias_v[...]

        def fire(g):
            return pltpu.async_copy(
                tw_hbm.at[idx_v.at[pl.ds(g * CHUNK, CHUNK)]],
                rows[g % NBUF],
                sems[g % NBUF],
            )

        handles = [fire(g) for g in range(NBUF - 1)]
        zero = jnp.zeros((PAD_D,), jnp.float32)
        acc = (zero, zero, zero, zero)
        for g in range(CPW):
            if g + NBUF - 1 < CPW:
                handles.append(fire(g + NBUF - 1))
            handles[g].wait()
            rbuf = rows[g % NBUF]

            def body(r, a):
                return (
                    a[0] + rbuf[r * 4],
                    a[1] + rbuf[r * 4 + 1],
                    a[2] + rbuf[r * 4 + 2],
                    a[3] + rbuf[r * 4 + 3],
                )

            acc = lax.fori_loop(0, CHUNK // 4, body, acc, unroll=4)
            if g % CPE == CPE - 1:
                e = g // CPE
                tot = (acc[0] + acc[1]) + (acc[2] + acc[3])
                cnt = jnp.broadcast_to(tot[2], (PAD_D,))
                res = tot / cnt + bias_vec
                out_buf[pl.ds(e * PAD_D, PAD_D)] = res
                acc = (zero, zero, zero, zero)
        pltpu.sync_copy(
            out_buf,
            out_hbm.at[pl.ds(wid * (EX_PER_W * PAD_D), EX_PER_W * PAD_D)],
        )

    return k(tablew, tok3, bias16)


def kernel(inputs, unigram, bigram, W, b):
    inputs = inputs.astype(jnp.int32)
    w_pad = jnp.zeros((EMBED, PAD_D), jnp.float32).at[:, :2].set(W)
    tablew = _tables_times_w(unigram.T, bigram.T, w_pad).reshape(
        N_TOT * 8, PAD_D
    )
    tok = jnp.pad(inputs, ((0, 0), (0, SEQ_PAD - SEQ)))
    tok = jnp.where(tok < UNI, tok, tok + IDX_OFF) * 8
    tok2 = tok.reshape(NW, CPW * CHUNK)
    b16 = jnp.zeros((PAD_D,), jnp.float32).at[:2].set(b)
    out = _pool(tablew, tok2, b16)
    return out.reshape(BATCH, PAD_D)[:, :2]


# speedup vs baseline: 1.2917x; 1.2803x over previous
import functools

import jax
import jax.numpy as jnp
from jax import lax
from jax.experimental import pallas as pl
from jax.experimental.pallas import tpu as pltpu
from jax.experimental.pallas import tpu_sc as plsc

UNI = 100000
BUCKETS = 800000
EMBED = 300
BATCH = 1024
SEQ = 500
PAD_D = 16
SEQ_PAD = 512
CHUNK = 512
NC, NS = 2, 16
NW = NC * NS
EX_PER_W = BATCH // NW
CPE = SEQ_PAD // CHUNK
CPW = EX_PER_W * CPE
NBUF = 4

MM_BLK = 8192
SLAB = MM_BLK // 8
NU_BLKS = (UNI + MM_BLK - 1) // MM_BLK
BI_BASE = NU_BLKS * MM_BLK
IDX_OFF = BI_BASE - UNI
N_TOT = BI_BASE + BUCKETS
MM_GRID = (N_TOT + MM_BLK - 1) // MM_BLK
N_PAD = MM_GRID * MM_BLK


def _mm_body(xu_ref, xb_ref, w_ref, o_ref):
    pid = pl.program_id(0)

    def emit(x_ref, zero_tail):
        yt = lax.dot_general(
            w_ref[...],
            x_ref[...],
            (((0,), (0,)), ((), ())),
            preferred_element_type=jnp.float32,
        )
        col = lax.broadcasted_iota(jnp.int32, yt.shape, 0)
        yt = jnp.where(col == 2, 1.0, yt)
        if zero_tail:
            row = pid * MM_BLK + lax.broadcasted_iota(jnp.int32, yt.shape, 1)
            yt = jnp.where((row == 0) | (row >= UNI), 0.0, yt)
        alt = jnp.concatenate(
            [yt[:, s * SLAB : (s + 1) * SLAB] for s in range(8)], axis=0
        )
        o_ref[...] = alt.T

    @pl.when(pid < NU_BLKS)
    def _():
        emit(xu_ref, True)

    @pl.when(pid >= NU_BLKS)
    def _():
        emit(xb_ref, False)


def _tables_times_w(uni_t, bi_t, w_pad):
    return pl.pallas_call(
        _mm_body,
        grid=(MM_GRID,),
        in_specs=[
            pl.BlockSpec(
                (EMBED, MM_BLK), lambda i: (0, jnp.minimum(i, NU_BLKS - 1))
            ),
            pl.BlockSpec(
                (EMBED, MM_BLK), lambda i: (0, jnp.maximum(i - NU_BLKS, 0))
            ),
            pl.BlockSpec((EMBED, PAD_D), lambda i: (0, 0)),
        ],
        out_specs=pl.BlockSpec((SLAB, 128), lambda i: (i, 0)),
        out_shape=jax.ShapeDtypeStruct((N_PAD // 8, 128), jnp.float32),
    )(uni_t, bi_t, w_pad)


def _pool(tablew, tok3, bias16):

    @functools.partial(
        pl.kernel,
        out_type=jax.ShapeDtypeStruct((BATCH * PAD_D,), jnp.float32),
        mesh=plsc.VectorSubcoreMesh(core_axis_name="c", subcore_axis_name="s"),
        compiler_params=pltpu.CompilerParams(use_tc_tiling_on_sc=False),
        scratch_types=(
            [
                pltpu.VMEM((CPW * CHUNK,), jnp.int32),
                pltpu.VMEM((EX_PER_W * PAD_D,), jnp.float32),
                pltpu.VMEM((PAD_D,), jnp.float32),
            ]
            + [pltpu.VMEM((CHUNK, PAD_D), jnp.float32) for _ in range(NBUF)]
            + [pltpu.SemaphoreType.DMA for _ in range(NBUF)]
        ),
    )
    def k(tw_hbm, tok_hbm, bias_hbm, out_hbm, idx_v, out_buf, bias_v, *rest):
        rows = rest[:NBUF]
        sems = rest[NBUF:]
        wid = lax.axis_index("s") * NC + lax.axis_index("c")
        pltpu.sync_copy(tok_hbm.at[wid], idx_v)
        pltpu.sync_copy(bias_hbm, bias_v)
        bias_vec = bias_v[...]

        def fire(g):
            return pltpu.async_copy(
                tw_hbm.at[idx_v.at[pl.ds(g * CHUNK, CHUNK)]],
                rows[g % NBUF],
                sems[g % NBUF],
            )

        handles = [fire(g) for g in range(NBUF - 1)]
        zero = jnp.zeros((PAD_D,), jnp.float32)
        acc = (zero, zero, zero, zero)
        for g in range(CPW):
            if g + NBUF - 1 < CPW:
                handles.append(fire(g + NBUF - 1))
            handles[g].wait()
            rbuf = rows[g % NBUF]

            def body(r, a):
                return (
                    a[0] + rbuf[r * 4],
                    a[1] + rbuf[r * 4 + 1],
                    a[2] + rbuf[r * 4 + 2],
                    a[3] + rbuf[r * 4 + 3],
                )

            acc = lax.fori_loop(0, CHUNK // 4, body, acc, unroll=4)
            if g % CPE == CPE - 1:
                e = g // CPE
                tot = (acc[0] + acc[1]) + (acc[2] + acc[3])
                cnt = jnp.broadcast_to(tot[2], (PAD_D,))
                res = tot / cnt + bias_vec
                out_buf[pl.ds(e * PAD_D, PAD_D)] = res
                acc = (zero, zero, zero, zero)
        pltpu.sync_copy(
            out_buf,
            out_hbm.at[pl.ds(wid * (EX_PER_W * PAD_D), EX_PER_W * PAD_D)],
        )

    return k(tablew, tok3, bias16)


def kernel(inputs, unigram, bigram, W, b):
    inputs = inputs.astype(jnp.int32)
    w_pad = jnp.zeros((EMBED, PAD_D), jnp.float32).at[:, :2].set(W)
    tablew = _tables_times_w(unigram.T, bigram.T, w_pad).reshape(N_PAD, PAD_D)
    tok = jnp.pad(inputs, ((0, 0), (0, SEQ_PAD - SEQ)))
    m = jnp.where(tok < UNI, tok, tok + IDX_OFF)
    tok = (m // MM_BLK) * MM_BLK + 8 * (m % SLAB) + (m % MM_BLK) // SLAB
    tok2 = tok.reshape(NW, CPW * CHUNK)
    b16 = jnp.zeros((PAD_D,), jnp.float32).at[:2].set(b)
    out = _pool(tablew, tok2, b16)
    return out.reshape(BATCH, PAD_D)[:, :2]
